# baseline (speedup 1.0000x reference)
import functools

import jax
import jax.numpy as jnp
from jax import lax
from jax.experimental import pallas as pl
from jax.experimental.pallas import tpu as pltpu
from jax.experimental.pallas import tpu_sc as plsc

N = 10000
D = 128
E = 320000
NC, NS = 2, 16
NW = NC * NS
CH = 128
NCH = 80
EPW = NCH * CH
EPAD = EPW * NW
NP = 10112
RPT = NP // NS
BR = 1000

_mesh = plsc.VectorSubcoreMesh(core_axis_name="c", subcore_axis_name="s")


def _deg_body(dst3, zerosD, onesD, out, deg_sh, idx_v, ones_v, dsem):
    c = lax.axis_index("c")
    s = lax.axis_index("s")
    wid = s * NC + c
    r0 = s * RPT
    pltpu.sync_copy(zerosD.at[pl.ds(r0, RPT)], deg_sh.at[pl.ds(r0, RPT)])
    pltpu.sync_copy(onesD, ones_v)
    pltpu.sync_copy(dst3.at[wid], idx_v)
    plsc.subcore_barrier()

    K = 8

    def fire(j):
        pltpu.async_copy(ones_v, deg_sh.at[idx_v.at[j]], dsem, add=True)

    def drain():
        pltpu.make_async_copy(ones_v, deg_sh.at[idx_v.at[0]], dsem).wait()

    def prol(j, carry):
        fire(j)
        return carry

    def body(j, carry):
        fire(j + K)
        drain()
        return carry

    def epil(j, carry):
        drain()
        return carry

    lax.fori_loop(0, K, prol, 0)
    lax.fori_loop(0, NCH - K, body, 0)
    lax.fori_loop(0, K, epil, 0)
    plsc.subcore_barrier()
    pltpu.sync_copy(deg_sh.at[pl.ds(r0, RPT)], out.at[pl.ds(c * NP + r0, RPT)])


_deg_call = pl.kernel(
    _deg_body,
    out_type=jax.ShapeDtypeStruct((NC * NP, D), jnp.float32),
    mesh=_mesh,
    scratch_types=[
        pltpu.VMEM_SHARED((NP, D), jnp.float32),
        pltpu.VMEM((NCH, CH), jnp.int32),
        pltpu.VMEM((CH, D), jnp.float32),
        pltpu.SemaphoreType.DMA,
    ],
)


NH = NCH // 2
NQ = NH // 2


def _scat_body(table, q3, zerosD, out, acc_sh, q0, q1, rows0, rows1,
               i0, i1, g0, g1, s0, s1):
    c = lax.axis_index("c")
    s = lax.axis_index("s")
    wid = s * NC + c
    r0 = s * RPT
    pltpu.sync_copy(zerosD.at[pl.ds(r0, RPT)], acc_sh.at[pl.ds(r0, RPT)])
    plsc.subcore_barrier()

    def load_idx(p, q, sem):
        pltpu.async_copy(q3.at[wid].at[p], q, sem)

    def load_idx_wait(q, sem):
        pltpu.make_async_copy(q3.at[wid].at[0], q, sem).wait()

    def gather(q, row, buf, sem):
        pltpu.async_copy(table.at[q.at[row]], buf, sem)

    def gather_wait(q, row, buf, sem):
        pltpu.make_async_copy(table.at[q.at[row]], buf, sem).wait()

    def scat(buf, q, row, sem):
        pltpu.async_copy(buf, acc_sh.at[q.at[row]], sem, add=True)

    def scat_wait(buf, q, row, sem):
        pltpu.make_async_copy(buf, acc_sh.at[q.at[row]], sem).wait()

    def half(p_next, qa, qb, ia, ib):
        gather_wait(qa, 0, rows0, g0)
        scat(rows0, qa, 1, s0)
        gather_wait(qa, 2, rows1, g1)
        scat(rows1, qa, 3, s1)
        pltpu.make_async_copy(q3.at[wid].at[0], qb, ib).wait()
        scat_wait(rows0, qa, 1, s0)
        gather(qb, 0, rows0, g0)
        scat_wait(rows1, qa, 3, s1)
        gather(qb, 2, rows1, g1)
        load_idx(p_next, qa, ia)

    load_idx(0, q0, i0)
    load_idx_wait(q0, i0)
    gather(q0, 0, rows0, g0)
    gather(q0, 2, rows1, g1)
    load_idx(1, q1, i1)

    def body(t, carry):
        half(2 * t + 2, q0, q1, i0, i1)
        half(2 * t + 3, q1, q0, i1, i0)
        return carry

    lax.fori_loop(0, NQ, body, 0)
    gather_wait(q0, 0, rows0, g0)
    gather_wait(q0, 2, rows1, g1)
    load_idx_wait(q1, i1)

    plsc.subcore_barrier()
    pltpu.sync_copy(acc_sh.at[pl.ds(r0, RPT)], out.at[pl.ds(c * NP + r0, RPT)])


_scat_call = pl.kernel(
    _scat_body,
    out_type=jax.ShapeDtypeStruct((NC * NP, D), jnp.float32),
    mesh=_mesh,
    scratch_types=[
        pltpu.VMEM_SHARED((NP, D), jnp.float32),
        pltpu.VMEM((4, CH), jnp.int32),
        pltpu.VMEM((4, CH), jnp.int32),
        pltpu.VMEM((CH, D), jnp.float32),
        pltpu.VMEM((CH, D), jnp.float32),
        pltpu.SemaphoreType.DMA,
        pltpu.SemaphoreType.DMA,
        pltpu.SemaphoreType.DMA,
        pltpu.SemaphoreType.DMA,
        pltpu.SemaphoreType.DMA,
        pltpu.SemaphoreType.DMA,
    ],
)


def _pre_body(x_ref, w_ref, d0_ref, d1_ref, hs_ref, dinv_ref):
    x0 = jnp.clip(x_ref[...], -100.0, 100.0)
    deg = d0_ref[...][:, 0:1] + d1_ref[...][:, 0:1] + 1.0
    dinv = lax.rsqrt(deg)
    h = jnp.dot(x0, w_ref[...], preferred_element_type=jnp.float32)
    hs_ref[...] = h * dinv
    dinv_ref[...] = jnp.broadcast_to(dinv, (BR, 16))


_pre_call = pl.pallas_call(
    _pre_body,
    grid=(N // BR,),
    in_specs=[
        pl.BlockSpec((BR, D), lambda i: (i, 0)),
        pl.BlockSpec((D, D), lambda i: (0, 0)),
        pl.BlockSpec((BR, D), lambda i: (i, 0)),
        pl.BlockSpec((BR, D), lambda i: (i, 0)),
    ],
    out_specs=[
        pl.BlockSpec((BR, D), lambda i: (i, 0)),
        pl.BlockSpec((BR, 16), lambda i: (i, 0)),
    ],
    out_shape=[
        jax.ShapeDtypeStruct((N, D), jnp.float32),
        jax.ShapeDtypeStruct((N, 16), jnp.float32),
    ],
)


def _mid_body(p0_ref, p1_ref, hs_ref, dinv_ref, b_ref, w_ref, out_ref):
    dinv = dinv_ref[...][:, 0:1]
    y = dinv * (p0_ref[...] + p1_ref[...] + hs_ref[...]) + b_ref[...]
    y = jnp.maximum(y, 0.0)
    out_ref[...] = jnp.dot(y, w_ref[...], preferred_element_type=jnp.float32) * dinv


_mid_call = pl.pallas_call(
    _mid_body,
    grid=(N // BR,),
    in_specs=[
        pl.BlockSpec((BR, D), lambda i: (i, 0)),
        pl.BlockSpec((BR, D), lambda i: (i, 0)),
        pl.BlockSpec((BR, D), lambda i: (i, 0)),
        pl.BlockSpec((BR, 16), lambda i: (i, 0)),
        pl.BlockSpec((1, D), lambda i: (0, 0)),
        pl.BlockSpec((D, D), lambda i: (0, 0)),
    ],
    out_specs=pl.BlockSpec((BR, D), lambda i: (i, 0)),
    out_shape=jax.ShapeDtypeStruct((N, D), jnp.float32),
)


def _fin_body(q0_ref, q1_ref, hs_ref, dinv_ref, b_ref, x_ref, wh_ref, wx_ref,
              bg_ref, out_ref):
    x0 = jnp.clip(x_ref[...], -100.0, 100.0)
    dinv = dinv_ref[...][:, 0:1]
    h2 = dinv * (q0_ref[...] + q1_ref[...] + hs_ref[...]) + b_ref[...]
    h = jnp.maximum(h2, 0.0) + x0
    g = jax.nn.sigmoid(
        jnp.dot(h, wh_ref[...], preferred_element_type=jnp.float32)
        + jnp.dot(x0, wx_ref[...], preferred_element_type=jnp.float32)
        + bg_ref[...]
    )
    out_ref[...] = g * h + (1.0 - g) * x0


_fin_call = pl.pallas_call(
    _fin_body,
    grid=(N // BR,),
    in_specs=[
        pl.BlockSpec((BR, D), lambda i: (i, 0)),
        pl.BlockSpec((BR, D), lambda i: (i, 0)),
        pl.BlockSpec((BR, D), lambda i: (i, 0)),
        pl.BlockSpec((BR, 16), lambda i: (i, 0)),
        pl.BlockSpec((1, D), lambda i: (0, 0)),
        pl.BlockSpec((BR, D), lambda i: (i, 0)),
        pl.BlockSpec((D, D), lambda i: (0, 0)),
        pl.BlockSpec((D, D), lambda i: (0, 0)),
        pl.BlockSpec((1, D), lambda i: (0, 0)),
    ],
    out_specs=pl.BlockSpec((BR, D), lambda i: (i, 0)),
    out_shape=jax.ShapeDtypeStruct((N, D), jnp.float32),
)


@jax.jit
def kernel(x, edge_index, W1, b1, W2, b2, Wg, bg):
    src = edge_index[0].astype(jnp.int32)
    dst = edge_index[1].astype(jnp.int32)
    pad = EPAD - E
    srcp = jnp.concatenate([src, jnp.zeros((pad,), jnp.int32)]).reshape(NW, NCH, CH)
    dstp = jnp.concatenate([dst, jnp.full((pad,), N, jnp.int32)]).reshape(NW, NCH, CH)
    sq = srcp.reshape(NW, NH, 2, CH)
    dq = dstp.reshape(NW, NH, 2, CH)
    q3 = jnp.stack([sq[:, :, 0], dq[:, :, 0], sq[:, :, 1], dq[:, :, 1]], axis=2)
    qpad = jnp.broadcast_to(
        jnp.array([0, N, 0, N], jnp.int32)[None, None, :, None], (NW, 2, 4, CH))
    q3 = jnp.concatenate([q3, qpad], axis=1)
    zerosD = jnp.zeros((NP, D), jnp.float32)
    onesD = jnp.ones((CH, D), jnp.float32)

    degp = _deg_call(dstp, zerosD, onesD)
    d0, d1 = degp[0:N], degp[NP:NP + N]

    hs1, dinv16 = _pre_call(x, W1, d0, d1)

    acc1 = _scat_call(hs1, q3, zerosD)
    hs2 = _mid_call(acc1[0:N], acc1[NP:NP + N], hs1, dinv16,
                    b1.reshape(1, D), W2)

    acc2 = _scat_call(hs2, q3, zerosD)
    out = _fin_call(acc2[0:N], acc2[NP:NP + N], hs2, dinv16,
                    b2.reshape(1, D), x, Wg[:D], Wg[D:], bg.reshape(1, D))
    return out


# speedup vs baseline: 1.3205x; 1.3205x over previous
import functools

import jax
import jax.numpy as jnp
from jax import lax
from jax.experimental import pallas as pl
from jax.experimental.pallas import tpu as pltpu
from jax.experimental.pallas import tpu_sc as plsc

N = 10000
D = 128
E = 320000
NC, NS = 2, 16
NW = NC * NS
CH = 128
NCH = 80
NCHP = NCH + 2
NH = NCH // 2
EPW = NCH * CH
EPAD = EPW * NW
NP = 10112
RPT = NP // NS
BR = 1000

_mesh = plsc.VectorSubcoreMesh(core_axis_name="c", subcore_axis_name="s")


def _deg_body(dst3, zerosD, onesD, out, deg_sh, idx_v, ones_v, dsem):
    c = lax.axis_index("c")
    s = lax.axis_index("s")
    wid = s * NC + c
    r0 = s * RPT
    pltpu.sync_copy(zerosD.at[pl.ds(r0, RPT)], deg_sh.at[pl.ds(r0, RPT)])
    pltpu.sync_copy(onesD, ones_v)
    pltpu.sync_copy(dst3.at[wid], idx_v)
    plsc.subcore_barrier()

    K = 8

    def fire(j):
        pltpu.async_copy(ones_v, deg_sh.at[idx_v.at[j]], dsem, add=True)

    def drain():
        pltpu.make_async_copy(ones_v, deg_sh.at[idx_v.at[0]], dsem).wait()

    def prol(j, carry):
        fire(j)
        return carry

    def body(j, carry):
        fire(j + K)
        drain()
        return carry

    def epil(j, carry):
        drain()
        return carry

    lax.fori_loop(0, K, prol, 0)
    lax.fori_loop(0, NCH - K, body, 0)
    lax.fori_loop(0, K, epil, 0)
    plsc.subcore_barrier()
    pltpu.sync_copy(deg_sh.at[pl.ds(r0, RPT)], out.at[pl.ds(c * NP + r0, RPT)])


_deg_call = pl.kernel(
    _deg_body,
    out_type=jax.ShapeDtypeStruct((NC * NP, D), jnp.float32),
    mesh=_mesh,
    scratch_types=[
        pltpu.VMEM_SHARED((NP, D), jnp.float32),
        pltpu.VMEM((NCHP, CH), jnp.int32),
        pltpu.VMEM((CH, D), jnp.float32),
        pltpu.SemaphoreType.DMA,
    ],
)


def _scat_body(table, src3, dstH, zerosD, out, acc_sh, sidx,
               rows0, rows1, db0, db1, g0, g1, s0, s1, i0, i1):
    c = lax.axis_index("c")
    s = lax.axis_index("s")
    wid = s * NC + c
    r0 = s * RPT
    pltpu.sync_copy(zerosD.at[pl.ds(r0, RPT)], acc_sh.at[pl.ds(r0, RPT)])
    pltpu.sync_copy(src3.at[wid], sidx)
    plsc.subcore_barrier()

    def gather(j, buf, sem):
        pltpu.async_copy(table.at[sidx.at[j]], buf, sem)

    def gather_wait(j, buf, sem):
        pltpu.make_async_copy(table.at[sidx.at[j]], buf, sem).wait()

    def dload(j, db, sem):
        pltpu.async_copy(dstH.at[wid * NCHP + j], db, sem)

    def dwait(db, sem):
        pltpu.make_async_copy(dstH.at[wid * NCHP], db, sem).wait()

    def scat(buf, db, sem):
        pltpu.async_copy(buf, acc_sh.at[db.at[0]], sem, add=True)

    def scat_wait(buf, db, sem):
        pltpu.make_async_copy(buf, acc_sh.at[db.at[0]], sem).wait()

    dload(0, db0, i0)
    gather(0, rows0, g0)
    gather_wait(0, rows0, g0)
    dwait(db0, i0)
    scat(rows0, db0, s0)
    gather(1, rows1, g1)
    dload(1, db1, i1)
    gather_wait(1, rows1, g1)
    scat_wait(rows0, db0, s0)
    dload(2, db0, i0)
    dwait(db1, i1)
    scat(rows1, db1, s1)
    gather(2, rows0, g0)

    def body(p, carry):
        j = 2 * p
        gather_wait(j, rows0, g0)
        scat_wait(rows1, db1, s1)
        dload(j + 1, db1, i1)
        dwait(db0, i0)
        scat(rows0, db0, s0)
        gather(j + 1, rows1, g1)
        gather_wait(j + 1, rows1, g1)
        scat_wait(rows0, db0, s0)
        dload(j + 2, db0, i0)
        dwait(db1, i1)
        scat(rows1, db1, s1)
        gather(j + 2, rows0, g0)
        return carry

    lax.fori_loop(1, NH, body, 0)
    gather_wait(NCH, rows0, g0)
    scat_wait(rows1, db1, s1)
    dwait(db0, i0)

    plsc.subcore_barrier()
    pltpu.sync_copy(acc_sh.at[pl.ds(r0, RPT)], out.at[pl.ds(c * NP + r0, RPT)])


_scat_call = pl.kernel(
    _scat_body,
    out_type=jax.ShapeDtypeStruct((NC * NP, D), jnp.float32),
    mesh=_mesh,
    scratch_types=[
        pltpu.VMEM_SHARED((NP, D), jnp.float32),
        pltpu.VMEM((NCHP, CH), jnp.int32),
        pltpu.VMEM((CH, D), jnp.float32),
        pltpu.VMEM((CH, D), jnp.float32),
        pltpu.VMEM((1, CH), jnp.int32),
        pltpu.VMEM((1, CH), jnp.int32),
        pltpu.SemaphoreType.DMA,
        pltpu.SemaphoreType.DMA,
        pltpu.SemaphoreType.DMA,
        pltpu.SemaphoreType.DMA,
        pltpu.SemaphoreType.DMA,
        pltpu.SemaphoreType.DMA,
    ],
)


def _prep_edges(src, dst):
    pad = EPAD - E
    srcp = jnp.concatenate([src, jnp.zeros((pad,), jnp.int32)]).reshape(NW, NCH, CH)
    dstp = jnp.concatenate([dst, jnp.full((pad,), N, jnp.int32)]).reshape(NW, NCH, CH)
    sdum = jnp.zeros((NW, NCHP - NCH, CH), jnp.int32)
    ddum = jnp.full((NW, NCHP - NCH, CH), N, jnp.int32)
    src3 = jnp.concatenate([srcp, sdum], axis=1)
    dst3 = jnp.concatenate([dstp, ddum], axis=1)
    dstH = dst3.reshape(NW * NCHP, 1, CH)
    return src3, dst3, dstH


def _pre_body(x_ref, w_ref, d0_ref, d1_ref, hs_ref, dinv_ref):
    x0 = jnp.clip(x_ref[...], -100.0, 100.0)
    deg = d0_ref[...][:, 0:1] + d1_ref[...][:, 0:1] + 1.0
    dinv = lax.rsqrt(deg)
    h = jnp.dot(x0, w_ref[...], preferred_element_type=jnp.float32)
    hs_ref[...] = h * dinv
    dinv_ref[...] = jnp.broadcast_to(dinv, (BR, 16))


_pre_call = pl.pallas_call(
    _pre_body,
    grid=(N // BR,),
    in_specs=[
        pl.BlockSpec((BR, D), lambda i: (i, 0)),
        pl.BlockSpec((D, D), lambda i: (0, 0)),
        pl.BlockSpec((BR, D), lambda i: (i, 0)),
        pl.BlockSpec((BR, D), lambda i: (i, 0)),
    ],
    out_specs=[
        pl.BlockSpec((BR, D), lambda i: (i, 0)),
        pl.BlockSpec((BR, 16), lambda i: (i, 0)),
    ],
    out_shape=[
        jax.ShapeDtypeStruct((N, D), jnp.float32),
        jax.ShapeDtypeStruct((N, 16), jnp.float32),
    ],
)


def _mid_body(p0_ref, p1_ref, hs_ref, dinv_ref, b_ref, w_ref, out_ref):
    dinv = dinv_ref[...][:, 0:1]
    y = dinv * (p0_ref[...] + p1_ref[...] + hs_ref[...]) + b_ref[...]
    y = jnp.maximum(y, 0.0)
    out_ref[...] = jnp.dot(y, w_ref[...], preferred_element_type=jnp.float32) * dinv


_mid_call = pl.pallas_call(
    _mid_body,
    grid=(N // BR,),
    in_specs=[
        pl.BlockSpec((BR, D), lambda i: (i, 0)),
        pl.BlockSpec((BR, D), lambda i: (i, 0)),
        pl.BlockSpec((BR, D), lambda i: (i, 0)),
        pl.BlockSpec((BR, 16), lambda i: (i, 0)),
        pl.BlockSpec((1, D), lambda i: (0, 0)),
        pl.BlockSpec((D, D), lambda i: (0, 0)),
    ],
    out_specs=pl.BlockSpec((BR, D), lambda i: (i, 0)),
    out_shape=jax.ShapeDtypeStruct((N, D), jnp.float32),
)


def _fin_body(q0_ref, q1_ref, hs_ref, dinv_ref, b_ref, x_ref, wh_ref, wx_ref,
              bg_ref, out_ref):
    x0 = jnp.clip(x_ref[...], -100.0, 100.0)
    dinv = dinv_ref[...][:, 0:1]
    h2 = dinv * (q0_ref[...] + q1_ref[...] + hs_ref[...]) + b_ref[...]
    h = jnp.maximum(h2, 0.0) + x0
    g = jax.nn.sigmoid(
        jnp.dot(h, wh_ref[...], preferred_element_type=jnp.float32)
        + jnp.dot(x0, wx_ref[...], preferred_element_type=jnp.float32)
        + bg_ref[...]
    )
    out_ref[...] = g * h + (1.0 - g) * x0


_fin_call = pl.pallas_call(
    _fin_body,
    grid=(N // BR,),
    in_specs=[
        pl.BlockSpec((BR, D), lambda i: (i, 0)),
        pl.BlockSpec((BR, D), lambda i: (i, 0)),
        pl.BlockSpec((BR, D), lambda i: (i, 0)),
        pl.BlockSpec((BR, 16), lambda i: (i, 0)),
        pl.BlockSpec((1, D), lambda i: (0, 0)),
        pl.BlockSpec((BR, D), lambda i: (i, 0)),
        pl.BlockSpec((D, D), lambda i: (0, 0)),
        pl.BlockSpec((D, D), lambda i: (0, 0)),
        pl.BlockSpec((1, D), lambda i: (0, 0)),
    ],
    out_specs=pl.BlockSpec((BR, D), lambda i: (i, 0)),
    out_shape=jax.ShapeDtypeStruct((N, D), jnp.float32),
)


@jax.jit
def kernel(x, edge_index, W1, b1, W2, b2, Wg, bg):
    src = edge_index[0].astype(jnp.int32)
    dst = edge_index[1].astype(jnp.int32)
    src3, dst3, dstH = _prep_edges(src, dst)
    zerosD = jnp.zeros((NP, D), jnp.float32)
    onesD = jnp.ones((CH, D), jnp.float32)

    degp = _deg_call(dst3, zerosD, onesD)
    d0, d1 = degp[0:N], degp[NP:NP + N]

    hs1, dinv16 = _pre_call(x, W1, d0, d1)

    acc1 = _scat_call(hs1, src3, dstH, zerosD)
    hs2 = _mid_call(acc1[0:N], acc1[NP:NP + N], hs1, dinv16,
                    b1.reshape(1, D), W2)

    acc2 = _scat_call(hs2, src3, dstH, zerosD)
    out = _fin_call(acc2[0:N], acc2[NP:NP + N], hs2, dinv16,
                    b2.reshape(1, D), x, Wg[:D], Wg[D:], bg.reshape(1, D))
    return out


# speedup vs baseline: 3.0974x; 2.3457x over previous
import functools

import jax
import jax.numpy as jnp
from jax import lax
from jax.experimental import pallas as pl
from jax.experimental.pallas import tpu as pltpu
from jax.experimental.pallas import tpu_sc as plsc

N = 10000
D = 128
E = 320000
NC, NS = 2, 16
NW = NC * NS
CH = 128
NP = 10112
RPT = NP // NS
BR = 1000

NCHD = 79
EPADD = NW * NCHD * CH

NCH0, NCH1 = 94, 63
NCHM = max(NCH0, NCH1)
E0 = NS * NCH0 * CH
E1CAP = NS * NCH1 * CH

_mesh = plsc.VectorSubcoreMesh(core_axis_name="c", subcore_axis_name="s")


def _deg_body(dst3, zerosD, onesD, out, deg_sh, idx_v, ones_v, dsem):
    c = lax.axis_index("c")
    s = lax.axis_index("s")
    wid = s * NC + c
    r0 = s * RPT
    pltpu.sync_copy(zerosD.at[pl.ds(r0, RPT)], deg_sh.at[pl.ds(r0, RPT)])
    pltpu.sync_copy(onesD, ones_v)
    pltpu.sync_copy(dst3.at[wid], idx_v)
    plsc.subcore_barrier()

    K = 8

    def fire(j):
        pltpu.async_copy(ones_v, deg_sh.at[idx_v.at[j]], dsem, add=True)

    def drain():
        pltpu.make_async_copy(ones_v, deg_sh.at[idx_v.at[0]], dsem).wait()

    def prol(j, carry):
        fire(j)
        return carry

    def body(j, carry):
        fire(j + K)
        drain()
        return carry

    def epil(j, carry):
        drain()
        return carry

    lax.fori_loop(0, K, prol, 0)
    lax.fori_loop(0, NCHD - K, body, 0)
    lax.fori_loop(0, K, epil, 0)
    plsc.subcore_barrier()
    pltpu.sync_copy(deg_sh.at[pl.ds(r0, RPT)], out.at[pl.ds(c * NP + r0, RPT)])


_deg_call = pl.kernel(
    _deg_body,
    out_type=jax.ShapeDtypeStruct((NC * NP, D), jnp.float32),
    mesh=_mesh,
    scratch_types=[
        pltpu.VMEM_SHARED((NP, D), jnp.float32),
        pltpu.VMEM((NCHD, CH), jnp.int32),
        pltpu.VMEM((CH, D), jnp.float32),
        pltpu.SemaphoreType.DMA,
    ],
)


def _scat_body(table, src3, dst3, zerosD, out, acc_sh, sidx, didx, rows, gsem):
    c = lax.axis_index("c")
    s = lax.axis_index("s")
    wid = s * NC + c
    r0 = s * RPT
    pltpu.sync_copy(zerosD.at[pl.ds(r0, RPT)], acc_sh.at[pl.ds(r0, RPT)])
    pltpu.sync_copy(src3.at[wid], sidx)
    pltpu.sync_copy(dst3.at[wid], didx)
    plsc.subcore_barrier()

    nch = jnp.where(c == 0, NCH0, NCH1)

    def body(j, carry):
        pltpu.async_copy(table.at[sidx.at[j]], rows, gsem).wait()
        pltpu.sync_copy(rows, acc_sh.at[didx.at[j]], add=True)
        return carry

    lax.fori_loop(0, nch, body, 0)
    plsc.subcore_barrier()
    pltpu.sync_copy(acc_sh.at[pl.ds(r0, RPT)], out.at[pl.ds(c * NP + r0, RPT)])


_scat_call = pl.kernel(
    _scat_body,
    out_type=jax.ShapeDtypeStruct((NC * NP, D), jnp.float32),
    mesh=_mesh,
    scratch_types=[
        pltpu.VMEM_SHARED((NP, D), jnp.float32),
        pltpu.VMEM((NCHM, CH), jnp.int32),
        pltpu.VMEM((NCHM, CH), jnp.int32),
        pltpu.VMEM((CH, D), jnp.float32),
        pltpu.SemaphoreType.DMA,
    ],
)


def _pad_to(a, n, fill):
    return jnp.concatenate([a, jnp.full((n - a.shape[0],), fill, a.dtype)])


def _core_slab(flat, nch, fill):
    a = flat.reshape(NS, nch, CH)
    if nch < NCHM:
        dum = jnp.full((NS, NCHM - nch, CH), fill, flat.dtype)
        a = jnp.concatenate([a, dum], axis=1)
    return a


def _prep_edges(src, dst):
    dd = _pad_to(dst, EPADD, N).reshape(NW, NCHD, CH)
    s0 = _core_slab(src[:E0], NCH0, 0)
    d0 = _core_slab(dst[:E0], NCH0, N)
    s1 = _core_slab(_pad_to(src[E0:], E1CAP, 0), NCH1, 0)
    d1 = _core_slab(_pad_to(dst[E0:], E1CAP, N), NCH1, N)
    src3 = jnp.stack([s0, s1], axis=1).reshape(NW, NCHM, CH)
    dst3 = jnp.stack([d0, d1], axis=1).reshape(NW, NCHM, CH)
    return dd, src3, dst3


def _pre_body(x_ref, w_ref, d0_ref, d1_ref, hs_ref, dinv_ref):
    x0 = jnp.clip(x_ref[...], -100.0, 100.0)
    deg = d0_ref[...][:, 0:1] + d1_ref[...][:, 0:1] + 1.0
    dinv = lax.rsqrt(deg)
    h = jnp.dot(x0, w_ref[...], preferred_element_type=jnp.float32)
    hs_ref[...] = h * dinv
    dinv_ref[...] = jnp.broadcast_to(dinv, (BR, 16))


_pre_call = pl.pallas_call(
    _pre_body,
    grid=(N // BR,),
    in_specs=[
        pl.BlockSpec((BR, D), lambda i: (i, 0)),
        pl.BlockSpec((D, D), lambda i: (0, 0)),
        pl.BlockSpec((BR, D), lambda i: (i, 0)),
        pl.BlockSpec((BR, D), lambda i: (i, 0)),
    ],
    out_specs=[
        pl.BlockSpec((BR, D), lambda i: (i, 0)),
        pl.BlockSpec((BR, 16), lambda i: (i, 0)),
    ],
    out_shape=[
        jax.ShapeDtypeStruct((N, D), jnp.float32),
        jax.ShapeDtypeStruct((N, 16), jnp.float32),
    ],
)


def _mid_body(p0_ref, p1_ref, hs_ref, dinv_ref, b_ref, w_ref, out_ref):
    dinv = dinv_ref[...][:, 0:1]
    y = dinv * (p0_ref[...] + p1_ref[...] + hs_ref[...]) + b_ref[...]
    y = jnp.maximum(y, 0.0)
    out_ref[...] = jnp.dot(y, w_ref[...], preferred_element_type=jnp.float32) * dinv


_mid_call = pl.pallas_call(
    _mid_body,
    grid=(N // BR,),
    in_specs=[
        pl.BlockSpec((BR, D), lambda i: (i, 0)),
        pl.BlockSpec((BR, D), lambda i: (i, 0)),
        pl.BlockSpec((BR, D), lambda i: (i, 0)),
        pl.BlockSpec((BR, 16), lambda i: (i, 0)),
        pl.BlockSpec((1, D), lambda i: (0, 0)),
        pl.BlockSpec((D, D), lambda i: (0, 0)),
    ],
    out_specs=pl.BlockSpec((BR, D), lambda i: (i, 0)),
    out_shape=jax.ShapeDtypeStruct((N, D), jnp.float32),
)


def _fin_body(q0_ref, q1_ref, hs_ref, dinv_ref, b_ref, x_ref, wh_ref, wx_ref,
              bg_ref, out_ref):
    x0 = jnp.clip(x_ref[...], -100.0, 100.0)
    dinv = dinv_ref[...][:, 0:1]
    h2 = dinv * (q0_ref[...] + q1_ref[...] + hs_ref[...]) + b_ref[...]
    h = jnp.maximum(h2, 0.0) + x0
    g = jax.nn.sigmoid(
        jnp.dot(h, wh_ref[...], preferred_element_type=jnp.float32)
        + jnp.dot(x0, wx_ref[...], preferred_element_type=jnp.float32)
        + bg_ref[...]
    )
    out_ref[...] = g * h + (1.0 - g) * x0


_fin_call = pl.pallas_call(
    _fin_body,
    grid=(N // BR,),
    in_specs=[
        pl.BlockSpec((BR, D), lambda i: (i, 0)),
        pl.BlockSpec((BR, D), lambda i: (i, 0)),
        pl.BlockSpec((BR, D), lambda i: (i, 0)),
        pl.BlockSpec((BR, 16), lambda i: (i, 0)),
        pl.BlockSpec((1, D), lambda i: (0, 0)),
        pl.BlockSpec((BR, D), lambda i: (i, 0)),
        pl.BlockSpec((D, D), lambda i: (0, 0)),
        pl.BlockSpec((D, D), lambda i: (0, 0)),
        pl.BlockSpec((1, D), lambda i: (0, 0)),
    ],
    out_specs=pl.BlockSpec((BR, D), lambda i: (i, 0)),
    out_shape=jax.ShapeDtypeStruct((N, D), jnp.float32),
)


@jax.jit
def kernel(x, edge_index, W1, b1, W2, b2, Wg, bg):
    src = edge_index[0].astype(jnp.int32)
    dst = edge_index[1].astype(jnp.int32)
    dd, src3, dst3 = _prep_edges(src, dst)
    zerosD = jnp.zeros((NP, D), jnp.float32)
    onesD = jnp.ones((CH, D), jnp.float32)

    degp = _deg_call(dd, zerosD, onesD)
    d0, d1 = degp[0:N], degp[NP:NP + N]

    hs1, dinv16 = _pre_call(x, W1, d0, d1)

    acc1 = _scat_call(hs1, src3, dst3, zerosD)
    hs2 = _mid_call(acc1[0:N], acc1[NP:NP + N], hs1, dinv16,
                    b1.reshape(1, D), W2)

    acc2 = _scat_call(hs2, src3, dst3, zerosD)
    out = _fin_call(acc2[0:N], acc2[NP:NP + N], hs2, dinv16,
                    b2.reshape(1, D), x, Wg[:D], Wg[D:], bg.reshape(1, D))
    return out
